# baseline (speedup 1.0000x reference)
import numpy as np
import jax
import jax.numpy as jnp
from jax.experimental import pallas as pl
from jax.experimental.pallas import tpu as pltpu

_B, _N, _H = 8, 100, 128
_HA, _HC = 64, 64
_P = _N * _N
_G = _B // 2
_F32 = jnp.float32


def _dot(a, b):
    return jnp.dot(a, b, preferred_element_type=_F32)


def _np_r():
    r = np.zeros((_P, _H), dtype=np.float32)
    p = np.arange(_P)
    r[p, p // _N] = 1.0
    return r


def _bd(w):
    z = jnp.zeros_like(w)
    return jnp.concatenate(
        [jnp.concatenate([w, z], axis=1), jnp.concatenate([z, w], axis=1)],
        axis=0)


def _extractor_kernel(nodes_ref, gv_ref, mask2f_ref, r_ref,
                      aw0g_ref, aw0a_ref, aw0c_ref, ab0_ref,
                      aw1_ref, ab1_ref, aw2_ref, ab2_ref, aw3_ref, ab3_ref,
                      cw0_ref, cb0_ref, cw1_ref, cb1_ref, cw2_ref, cb2_ref,
                      cw3_ref, cb3_ref,
                      out_ref, val_ref, dim_ref):
    @pl.when(pl.program_id(0) == 0)
    def _():
        gall = gv_ref[...]
        v = jnp.tanh(_dot(gall, cw0_ref[...]) + cb0_ref[...])
        v = jnp.tanh(_dot(v, cw1_ref[...]) + cb1_ref[...])
        v = jnp.tanh(_dot(v, cw2_ref[...]) + cb2_ref[...])
        val_ref[...] = _dot(v, cw3_ref[...]) + cb3_ref[...]

        valid_f = (mask2f_ref[...] == 1.0).astype(_F32)
        dim_ref[0, 0] = jnp.max(jnp.sum(valid_f, axis=2))
    dim = dim_ref[0, 0]

    p = pl.program_id(0)
    gv2 = gv_ref[pl.ds(2 * p, 2)]
    q = _dot(gv2, aw0g_ref[...]) + ab0_ref[...]
    g0 = jnp.concatenate([q[0:1], q[1:2]], axis=1)
    n0 = nodes_ref[0]
    n1 = nodes_ref[1]
    a0 = jnp.concatenate([_dot(n0, aw0a_ref[...]),
                          _dot(n1, aw0a_ref[...])], axis=1) + g0
    c0 = jnp.concatenate([_dot(n0, aw0c_ref[...]),
                          _dot(n1, aw0c_ref[...])], axis=1)
    c_rep = jnp.broadcast_to(c0[None], (_N, _N, 2 * _HA)) \
        .reshape(_P, 2 * _HA)
    a0p = jnp.concatenate([a0, jnp.zeros((_H - _N, 2 * _HA), _F32)],
                          axis=0)
    h = jnp.tanh(_dot(r_ref[...], a0p) + c_rep)
    b1 = jnp.concatenate([ab1_ref[...], ab1_ref[...]], axis=1)
    h = jnp.tanh(_dot(h, _bd(aw1_ref[...])) + b1)
    b2 = jnp.concatenate([ab2_ref[...], ab2_ref[...]], axis=1)
    h = jnp.tanh(_dot(h, _bd(aw2_ref[...])) + b2)
    logit2 = _dot(h, _bd(aw3_ref[...]))
    logit = jnp.swapaxes(logit2, 0, 1) + ab3_ref[0, 0]

    valid = mask2f_ref[p] == 1.0
    count2 = jnp.sum(valid.astype(_F32), axis=1, keepdims=True)
    neg_inf = jnp.full_like(logit, -jnp.inf)
    m = jnp.maximum(jnp.max(jnp.where(valid, logit, neg_inf),
                            axis=1, keepdims=True), 0.0)
    e = jnp.exp(logit - m)
    denom = (jnp.sum(jnp.where(valid, e, jnp.zeros_like(e)),
                     axis=1, keepdims=True)
             + (dim - count2) * jnp.exp(-m))
    out_ref[0] = jnp.where(valid, e / denom, jnp.zeros_like(e))


def kernel(embedded_features, aw0, ab0, aw1, ab1, aw2, ab2, aw3, ab3,
           cw0, cb0, cw1, cb1, cw2, cb2, cw3, cb3):
    gan = embedded_features[:, :, :_H]
    g = gan[:, 0, :]
    nodes = gan[:, 1:, :]
    mask = embedded_features[:, 1:, _H:]
    mask2 = mask.reshape(_G, 2, _P)

    r = jnp.asarray(_np_r())

    full = lambda shape: pl.BlockSpec(shape, lambda p: (0,) * len(shape))
    out2, val = pl.pallas_call(
        _extractor_kernel,
        grid=(_G,),
        in_specs=[
            pl.BlockSpec((2, _N, _H), lambda p: (p, 0, 0)),
            full((_B, _H)),
            full((_G, 2, _P)),
            full((_P, _H)),
            full((_H, _HA)), full((_H, _HA)), full((_H, _HA)),
            full((1, _HA)),
            full((_HA, _HA)), full((1, _HA)),
            full((_HA, _HA)), full((1, _HA)),
            full((_HA, 1)), full((1, 1)),
            full((_H, _HC)), full((1, _HC)),
            full((_HC, _HC)), full((1, _HC)),
            full((_HC, _HC)), full((1, _HC)),
            full((_HC, 1)), full((1, 1)),
        ],
        out_specs=[
            pl.BlockSpec((1, 2, _P), lambda p: (p, 0, 0)),
            full((_B, 1)),
        ],
        out_shape=[
            jax.ShapeDtypeStruct((_G, 2, _P), _F32),
            jax.ShapeDtypeStruct((_B, 1), _F32),
        ],
        scratch_shapes=[pltpu.SMEM((1, 1), _F32)],
    )(nodes, g, mask2, r,
      aw0[0:_H], aw0[_H:2 * _H], aw0[2 * _H:], ab0[None, :],
      aw1, ab1[None, :], aw2, ab2[None, :], aw3, ab3[None, :],
      cw0, cb0[None, :], cw1, cb1[None, :], cw2, cb2[None, :],
      cw3, cb3[None, :])

    filled = out2.reshape(_B, _P)
    return (filled, val.reshape(_B, 1, 1))
